# baseline (speedup 1.0000x reference)
import functools

import jax
import jax.numpy as jnp
from jax import lax
from jax.experimental import pallas as pl
from jax.experimental.pallas import tpu as pltpu
from jax.experimental.pallas import tpu_sc as plsc

N = 16384
D = 256
M = 512
S = 1024

NUM_CORES = 2
NUM_SUBCORES = 16
NUM_WORKERS = NUM_CORES * NUM_SUBCORES
SEGS_PER_WORKER = S // NUM_WORKERS
CHUNK = 128
LANES = 16
D_VECS = D // LANES


def _lower_bound_vec(seg_v, targets):
    pos = jnp.zeros((LANES,), jnp.int32)
    sz = N // 2
    while sz >= 1:
        probe = plsc.load_gather(seg_v, [pos + (sz - 1)])
        pos = jnp.where(probe < targets, pos + sz, pos)
        sz //= 2
    probe = plsc.load_gather(seg_v, [pos])
    return jnp.where(probe < targets, pos + 1, pos)


def _sc_body(data1_hbm, seg_hbm, agg_out, cnt_out,
             seg_v, buf_v, buf2_v, buf3_v, acc_v, cnt_v, b_sm,
             sem0, sem1, sem2, sem_seg):
    c = lax.axis_index("c")
    s = lax.axis_index("s")
    wid = c * NUM_SUBCORES + s
    s_lo = wid * SEGS_PER_WORKER

    seg_copy = pltpu.make_async_copy(seg_hbm, seg_v.at[pl.ds(0, N)], sem_seg)
    pltpu.async_copy(seg_hbm, seg_v.at[pl.ds(0, N)], sem_seg)

    zero = jnp.zeros((LANES,), jnp.float32)

    def zero_body(j, carry):
        for k in range(D_VECS):
            acc_v[j, pl.ds(k * LANES, LANES)] = zero
        return carry

    lax.fori_loop(0, SEGS_PER_WORKER, zero_body, 0, unroll=False)
    seg_copy.wait()

    lane = lax.iota(jnp.int32, LANES)
    lb0 = _lower_bound_vec(seg_v, s_lo + lane)
    lb1 = _lower_bound_vec(seg_v, s_lo + LANES + lane)
    ub0 = _lower_bound_vec(seg_v, s_lo + 1 + lane)
    ub1 = _lower_bound_vec(seg_v, s_lo + LANES + 1 + lane)

    r_lo = lb0[0]
    r_hi = ub1[LANES - 1]
    c_lo = lax.shift_right_logical(r_lo, 7)
    c_hi = lax.shift_right_logical(r_hi + (CHUNK - 1), 7)

    def process_chunk(ci, buf, carry):
        row0 = ci * CHUNK

        def seg_body(j, carry, _buf=buf, _row0=row0):
            lo = jnp.maximum(b_sm[j], _row0)
            hi = jnp.minimum(b_sm[j + 1], _row0 + CHUNK)

            def row_body(r, accs):
                rr = r - _row0
                return tuple(
                    accs[k] + _buf[rr, pl.ds(k * LANES, LANES)]
                    for k in range(D_VECS)
                )

            accs = lax.fori_loop(lo, hi, row_body, (zero,) * D_VECS,
                                 unroll=False)

            @pl.when(hi > lo)
            def _():
                for k in range(D_VECS):
                    plsc.addupdate(acc_v.at[j, pl.ds(k * LANES, LANES)],
                                   accs[k])

            return carry

        return lax.fori_loop(0, SEGS_PER_WORKER, seg_body, carry,
                             unroll=False)

    def issue(ci, buf, sem):
        pltpu.async_copy(data1_hbm.at[pl.ds(ci * CHUNK, CHUNK)], buf, sem)

    def drain(buf, sem):
        pltpu.make_async_copy(data1_hbm.at[pl.ds(0, CHUNK)], buf, sem).wait()

    rings = ((buf_v, sem0), (buf2_v, sem1), (buf3_v, sem2))
    for i, (buf, sem) in enumerate(rings):
        @pl.when(c_lo + i < c_hi)
        def _(i=i, buf=buf, sem=sem):
            issue(c_lo + i, buf, sem)

    for i in range(LANES):
        b_sm[i] = lb0[i]
        b_sm[LANES + i] = lb1[i]
    b_sm[SEGS_PER_WORKER] = ub1[LANES - 1]
    cnt_v[pl.ds(0, LANES)] = (ub0 - lb0).astype(jnp.float32)
    cnt_v[pl.ds(LANES, LANES)] = (ub1 - lb1).astype(jnp.float32)
    pltpu.sync_copy(cnt_v, cnt_out.at[pl.ds(s_lo, SEGS_PER_WORKER)])

    def trio_body(p, carry):
        c0 = c_lo + 3 * p
        for i, (buf, sem) in enumerate(rings):
            ci = c0 + i

            @pl.when(ci < c_hi)
            def _(buf=buf, sem=sem):
                drain(buf, sem)

            carry = process_chunk(ci, buf, carry)

            @pl.when(ci + 3 < c_hi)
            def _(ci=ci, buf=buf, sem=sem):
                issue(ci + 3, buf, sem)
        return carry

    ntrios = (c_hi - c_lo + 2) // 3
    lax.fori_loop(0, ntrios, trio_body, jnp.int32(0), unroll=False)

    pltpu.sync_copy(acc_v, agg_out.at[pl.ds(s_lo, SEGS_PER_WORKER)])


@functools.cache
def _sc_segment_sum():
    return pl.kernel(
        _sc_body,
        out_type=(
            jax.ShapeDtypeStruct((S, D), jnp.float32),
            jax.ShapeDtypeStruct((S,), jnp.float32),
        ),
        mesh=plsc.VectorSubcoreMesh(core_axis_name="c", subcore_axis_name="s",
                                    num_cores=NUM_CORES,
                                    num_subcores=NUM_SUBCORES),
        compiler_params=pltpu.CompilerParams(needs_layout_passes=False),
        scratch_types=[
            pltpu.VMEM((N + LANES,), jnp.int32),
            pltpu.VMEM((CHUNK, D), jnp.float32),
            pltpu.VMEM((CHUNK, D), jnp.float32),
            pltpu.VMEM((CHUNK, D), jnp.float32),
            pltpu.VMEM((SEGS_PER_WORKER, D), jnp.float32),
            pltpu.VMEM((SEGS_PER_WORKER,), jnp.float32),
            pltpu.SMEM((SEGS_PER_WORKER + 1,), jnp.int32),
            pltpu.SemaphoreType.DMA,
            pltpu.SemaphoreType.DMA,
            pltpu.SemaphoreType.DMA,
            pltpu.SemaphoreType.DMA,
        ],
    )


M_BLK = 128


def _tc_body(agg_ref, cnt_ref, d2_ref, out_ref):
    mat = lax.dot_general(agg_ref[...], d2_ref[...],
                          dimension_numbers=(((1,), (1,)), ((), ())),
                          preferred_element_type=jnp.float32)
    out_ref[...] = mat / jnp.maximum(cnt_ref[...], 1.0)


_tc_matmul = pl.pallas_call(
    _tc_body,
    grid=(M // M_BLK,),
    in_specs=[
        pl.BlockSpec((S, D), lambda m: (0, 0)),
        pl.BlockSpec((S, 1), lambda m: (0, 0)),
        pl.BlockSpec((M_BLK, D), lambda m: (m, 0)),
    ],
    out_specs=pl.BlockSpec((S, M_BLK), lambda m: (0, m)),
    out_shape=jax.ShapeDtypeStruct((S, M), jnp.float32),
)


@jax.jit
def kernel(data1, data2, segment_ids):
    seg = segment_ids.astype(jnp.int32)
    agg, cnt = _sc_segment_sum()(data1, seg)
    return _tc_matmul(agg, cnt.reshape(S, 1), data2)


# speedup vs baseline: 1.0345x; 1.0345x over previous
import functools

import jax
import jax.numpy as jnp
from jax import lax
from jax.experimental import pallas as pl
from jax.experimental.pallas import tpu as pltpu
from jax.experimental.pallas import tpu_sc as plsc

N = 16384
D = 256
M = 512
S = 1024

NUM_CORES = 2
NUM_SUBCORES = 16
NUM_WORKERS = NUM_CORES * NUM_SUBCORES
SEGS_PER_WORKER = S // NUM_WORKERS
CHUNK = 128
LANES = 16
D_VECS = D // LANES


def _lower_bound_vec(seg_v, targets):
    pos = jnp.zeros((LANES,), jnp.int32)
    sz = N // 2
    while sz >= 1:
        probe = plsc.load_gather(seg_v, [pos + (sz - 1)])
        pos = jnp.where(probe < targets, pos + sz, pos)
        sz //= 2
    probe = plsc.load_gather(seg_v, [pos])
    return jnp.where(probe < targets, pos + 1, pos)


def _sc_body(data1_hbm, seg_hbm, agg_out, cnt_out,
             seg_v, buf_v, buf2_v, buf3_v, acc_v, cnt_v, b_sm,
             sem0, sem1, sem2, sem_seg):
    c = lax.axis_index("c")
    s = lax.axis_index("s")
    wid = c * NUM_SUBCORES + s
    s_lo = wid * SEGS_PER_WORKER

    seg_copy = pltpu.make_async_copy(seg_hbm, seg_v.at[pl.ds(0, N)], sem_seg)
    pltpu.async_copy(seg_hbm, seg_v.at[pl.ds(0, N)], sem_seg)

    zero = jnp.zeros((LANES,), jnp.float32)

    def zero_body(j, carry):
        for k in range(D_VECS):
            acc_v[j, pl.ds(k * LANES, LANES)] = zero
        return carry

    lax.fori_loop(0, SEGS_PER_WORKER, zero_body, 0, unroll=False)
    seg_copy.wait()

    lane = lax.iota(jnp.int32, LANES)
    lb0 = _lower_bound_vec(seg_v, s_lo + lane)
    lb1 = _lower_bound_vec(seg_v, s_lo + LANES + lane)
    ub0 = _lower_bound_vec(seg_v, s_lo + 1 + lane)
    ub1 = _lower_bound_vec(seg_v, s_lo + LANES + 1 + lane)

    r_lo = lb0[0]
    r_hi = ub1[LANES - 1]
    c_lo = lax.shift_right_logical(r_lo, 7)
    c_hi = lax.shift_right_logical(r_hi + (CHUNK - 1), 7)

    def process_chunk(ci, buf, carry):
        row0 = ci * CHUNK

        def seg_body(j, carry, _buf=buf, _row0=row0):
            lo = jnp.maximum(b_sm[j], _row0)
            hi = jnp.minimum(b_sm[j + 1], _row0 + CHUNK)

            def row_body(r, accs):
                rr = r - _row0
                return tuple(
                    accs[k] + _buf[rr, pl.ds(k * LANES, LANES)]
                    for k in range(D_VECS)
                )

            accs = lax.fori_loop(lo, hi, row_body, (zero,) * D_VECS,
                                 unroll=False)

            @pl.when(hi > lo)
            def _():
                for k in range(D_VECS):
                    plsc.addupdate(acc_v.at[j, pl.ds(k * LANES, LANES)],
                                   accs[k])

            return carry

        return lax.fori_loop(0, SEGS_PER_WORKER, seg_body, carry,
                             unroll=False)

    def issue(ci, buf, sem):
        pltpu.async_copy(data1_hbm.at[pl.ds(ci * CHUNK, CHUNK)], buf, sem)

    def drain(buf, sem):
        pltpu.make_async_copy(data1_hbm.at[pl.ds(0, CHUNK)], buf, sem).wait()

    rings = ((buf_v, sem0), (buf2_v, sem1), (buf3_v, sem2))
    for i, (buf, sem) in enumerate(rings):
        @pl.when(c_lo + i < c_hi)
        def _(i=i, buf=buf, sem=sem):
            issue(c_lo + i, buf, sem)

    for i in range(LANES):
        b_sm[i] = lb0[i]
        b_sm[LANES + i] = lb1[i]
    b_sm[SEGS_PER_WORKER] = ub1[LANES - 1]
    cnt_v[pl.ds(0, LANES)] = (ub0 - lb0).astype(jnp.float32)
    cnt_v[pl.ds(LANES, LANES)] = (ub1 - lb1).astype(jnp.float32)
    pltpu.sync_copy(cnt_v, cnt_out.at[pl.ds(s_lo, SEGS_PER_WORKER)])

    def trio_body(p, carry):
        c0 = c_lo + 3 * p
        for i, (buf, sem) in enumerate(rings):
            ci = c0 + i

            @pl.when(ci < c_hi)
            def _(buf=buf, sem=sem):
                drain(buf, sem)

            carry = process_chunk(ci, buf, carry)

            @pl.when(ci + 3 < c_hi)
            def _(ci=ci, buf=buf, sem=sem):
                issue(ci + 3, buf, sem)
        return carry

    ntrios = (c_hi - c_lo + 2) // 3
    lax.fori_loop(0, ntrios, trio_body, jnp.int32(0), unroll=False)

    pltpu.sync_copy(acc_v, agg_out.at[pl.ds(s_lo, SEGS_PER_WORKER)])


@functools.cache
def _sc_segment_sum():
    return pl.kernel(
        _sc_body,
        out_type=(
            jax.ShapeDtypeStruct((S, D), jnp.float32),
            jax.ShapeDtypeStruct((S,), jnp.float32),
        ),
        mesh=plsc.VectorSubcoreMesh(core_axis_name="c", subcore_axis_name="s",
                                    num_cores=NUM_CORES,
                                    num_subcores=NUM_SUBCORES),
        compiler_params=pltpu.CompilerParams(needs_layout_passes=False),
        scratch_types=[
            pltpu.VMEM((N + LANES,), jnp.int32),
            pltpu.VMEM((CHUNK, D), jnp.float32),
            pltpu.VMEM((CHUNK, D), jnp.float32),
            pltpu.VMEM((CHUNK, D), jnp.float32),
            pltpu.VMEM((SEGS_PER_WORKER, D), jnp.float32),
            pltpu.VMEM((SEGS_PER_WORKER,), jnp.float32),
            pltpu.SMEM((SEGS_PER_WORKER + 1,), jnp.int32),
            pltpu.SemaphoreType.DMA,
            pltpu.SemaphoreType.DMA,
            pltpu.SemaphoreType.DMA,
            pltpu.SemaphoreType.DMA,
        ],
    )


M_BLK = 128


def _tc_body(agg_ref, cnt_ref, d2_ref, out_ref):
    mat = lax.dot_general(agg_ref[...], d2_ref[...],
                          dimension_numbers=(((1,), (1,)), ((), ())),
                          preferred_element_type=jnp.float32)
    out_ref[...] = mat / jnp.maximum(cnt_ref[...], 1.0)


_tc_matmul = pl.pallas_call(
    _tc_body,
    out_shape=jax.ShapeDtypeStruct((S, M), jnp.float32),
)


@jax.jit
def kernel(data1, data2, segment_ids):
    seg = segment_ids.astype(jnp.int32)
    agg, cnt = _sc_segment_sum()(data1, seg)
    return _tc_matmul(agg, cnt.reshape(S, 1), data2)


# speedup vs baseline: 1.0347x; 1.0002x over previous
import functools

import jax
import jax.numpy as jnp
from jax import lax
from jax.experimental import pallas as pl
from jax.experimental.pallas import tpu as pltpu
from jax.experimental.pallas import tpu_sc as plsc

N = 16384
D = 256
M = 512
S = 1024

NUM_CORES = 2
NUM_SUBCORES = 16
NUM_WORKERS = NUM_CORES * NUM_SUBCORES
SEGS_PER_WORKER = S // NUM_WORKERS
CHUNK = 128
LANES = 16
D_VECS = D // LANES


def _lower_bound_vec(seg_v, targets):
    pos = jnp.zeros((LANES,), jnp.int32)
    sz = N // 2
    while sz >= 1:
        probe = plsc.load_gather(seg_v, [pos + (sz - 1)])
        pos = jnp.where(probe < targets, pos + sz, pos)
        sz //= 2
    probe = plsc.load_gather(seg_v, [pos])
    return jnp.where(probe < targets, pos + 1, pos)


def _sc_body(data1_hbm, seg_hbm, agg_out, cnt_out,
             seg_v, buf_v, buf2_v, buf3_v, acc_v, cnt_v, b_sm,
             sem0, sem1, sem2, sem_seg):
    c = lax.axis_index("c")
    s = lax.axis_index("s")
    wid = c * NUM_SUBCORES + s
    s_lo = wid * SEGS_PER_WORKER

    seg_copy = pltpu.make_async_copy(seg_hbm, seg_v.at[pl.ds(0, N)], sem_seg)
    pltpu.async_copy(seg_hbm, seg_v.at[pl.ds(0, N)], sem_seg)

    zero = jnp.zeros((LANES,), jnp.float32)

    def zero_body(j, carry):
        for k in range(D_VECS):
            acc_v[j, pl.ds(k * LANES, LANES)] = zero
        return carry

    lax.fori_loop(0, SEGS_PER_WORKER, zero_body, 0, unroll=False)
    seg_copy.wait()

    lane = lax.iota(jnp.int32, LANES)
    lb0 = _lower_bound_vec(seg_v, s_lo + lane)
    lb1 = _lower_bound_vec(seg_v, s_lo + LANES + lane)
    ub0 = _lower_bound_vec(seg_v, s_lo + 1 + lane)
    ub1 = _lower_bound_vec(seg_v, s_lo + LANES + 1 + lane)

    r_lo = lb0[0]
    r_hi = ub1[LANES - 1]
    c_lo = lax.shift_right_logical(r_lo, 7)
    c_hi = lax.shift_right_logical(r_hi + (CHUNK - 1), 7)

    def process_chunk(ci, buf, carry):
        row0 = ci * CHUNK

        def seg_body(j, carry, _buf=buf, _row0=row0):
            lo = jnp.maximum(b_sm[j], _row0)
            hi = jnp.minimum(b_sm[j + 1], _row0 + CHUNK)

            def row_body(r, accs):
                rr = r - _row0
                return tuple(
                    accs[k] + _buf[rr, pl.ds(k * LANES, LANES)]
                    for k in range(D_VECS)
                )

            accs = lax.fori_loop(lo, hi, row_body, (zero,) * D_VECS,
                                 unroll=False)

            @pl.when(hi > lo)
            def _():
                for k in range(D_VECS):
                    plsc.addupdate(acc_v.at[j, pl.ds(k * LANES, LANES)],
                                   accs[k])

            return carry

        return lax.fori_loop(0, SEGS_PER_WORKER, seg_body, carry,
                             unroll=False)

    def issue(ci, buf, sem):
        pltpu.async_copy(data1_hbm.at[pl.ds(ci * CHUNK, CHUNK)], buf, sem)

    def drain(buf, sem):
        pltpu.make_async_copy(data1_hbm.at[pl.ds(0, CHUNK)], buf, sem).wait()

    rings = ((buf_v, sem0), (buf2_v, sem1), (buf3_v, sem2))
    for i, (buf, sem) in enumerate(rings):
        @pl.when(c_lo + i < c_hi)
        def _(i=i, buf=buf, sem=sem):
            issue(c_lo + i, buf, sem)

    for i in range(LANES):
        b_sm[i] = lb0[i]
        b_sm[LANES + i] = lb1[i]
    b_sm[SEGS_PER_WORKER] = ub1[LANES - 1]
    cnt_v[pl.ds(0, LANES)] = (ub0 - lb0).astype(jnp.float32)
    cnt_v[pl.ds(LANES, LANES)] = (ub1 - lb1).astype(jnp.float32)
    pltpu.sync_copy(cnt_v, cnt_out.at[pl.ds(s_lo, SEGS_PER_WORKER)])

    def trio_body(p, carry):
        c0 = c_lo + 3 * p
        for i, (buf, sem) in enumerate(rings):
            ci = c0 + i

            @pl.when(ci < c_hi)
            def _(buf=buf, sem=sem):
                drain(buf, sem)

            carry = process_chunk(ci, buf, carry)

            @pl.when(ci + 3 < c_hi)
            def _(ci=ci, buf=buf, sem=sem):
                issue(ci + 3, buf, sem)
        return carry

    ntrios = (c_hi - c_lo + 2) // 3
    lax.fori_loop(0, ntrios, trio_body, jnp.int32(0), unroll=False)

    pltpu.sync_copy(acc_v, agg_out.at[pl.ds(s_lo, SEGS_PER_WORKER)])


@functools.cache
def _sc_segment_sum():
    return pl.kernel(
        _sc_body,
        out_type=(
            jax.ShapeDtypeStruct((S, D), jnp.float32),
            jax.ShapeDtypeStruct((S,), jnp.float32),
        ),
        mesh=plsc.VectorSubcoreMesh(core_axis_name="c", subcore_axis_name="s",
                                    num_cores=NUM_CORES,
                                    num_subcores=NUM_SUBCORES),
        compiler_params=pltpu.CompilerParams(needs_layout_passes=False, skip_device_barrier=True),
        scratch_types=[
            pltpu.VMEM((N + LANES,), jnp.int32),
            pltpu.VMEM((CHUNK, D), jnp.float32),
            pltpu.VMEM((CHUNK, D), jnp.float32),
            pltpu.VMEM((CHUNK, D), jnp.float32),
            pltpu.VMEM((SEGS_PER_WORKER, D), jnp.float32),
            pltpu.VMEM((SEGS_PER_WORKER,), jnp.float32),
            pltpu.SMEM((SEGS_PER_WORKER + 1,), jnp.int32),
            pltpu.SemaphoreType.DMA,
            pltpu.SemaphoreType.DMA,
            pltpu.SemaphoreType.DMA,
            pltpu.SemaphoreType.DMA,
        ],
    )


M_BLK = 128


def _tc_body(agg_ref, cnt_ref, d2_ref, out_ref):
    mat = lax.dot_general(agg_ref[...], d2_ref[...],
                          dimension_numbers=(((1,), (1,)), ((), ())),
                          preferred_element_type=jnp.float32)
    out_ref[...] = mat / jnp.maximum(cnt_ref[...], 1.0)


_tc_matmul = pl.pallas_call(
    _tc_body,
    out_shape=jax.ShapeDtypeStruct((S, M), jnp.float32),
)


@jax.jit
def kernel(data1, data2, segment_ids):
    seg = segment_ids.astype(jnp.int32)
    agg, cnt = _sc_segment_sum()(data1, seg)
    return _tc_matmul(agg, cnt.reshape(S, 1), data2)
